# baseline (speedup 1.0000x reference)
import jax
import jax.numpy as jnp
from jax.experimental import pallas as pl
from jax.experimental.pallas import tpu as pltpu

N, C = 32768, 2048
BR = 512
G = N // BR


def _tc_loss(logits, target):
    def body(x_ref, t_ref, o_ref, acc):
        i = pl.program_id(0)

        @pl.when(i == 0)
        def _():
            acc[0, 0] = 0.0
            acc[0, 1] = 0.0

        x = x_ref[...]
        tb = t_ref[...]
        cols = jax.lax.broadcasted_iota(jnp.int32, (BR, C), 1)
        safe = jnp.where(tb >= 0, tb, -2)
        m = cols == safe[:, None]
        d = 1.0 - x
        s = jnp.sum(jnp.where(m, d * d, 0.0))
        cnt = jnp.sum(jnp.where(tb >= 0, 1.0, 0.0))
        acc[0, 0] += s
        acc[0, 1] += cnt

        @pl.when(i == G - 1)
        def _():
            o_ref[...] = jnp.full((1, 1), acc[0, 0] / acc[0, 1], jnp.float32)

    return pl.pallas_call(
        body,
        grid=(G,),
        in_specs=[
            pl.BlockSpec((BR, C), lambda i: (i, 0)),
            pl.BlockSpec((BR,), lambda i: (i,)),
        ],
        out_specs=pl.BlockSpec((1, 1), lambda i: (0, 0)),
        out_shape=jax.ShapeDtypeStruct((1, 1), jnp.float32),
        scratch_shapes=[pltpu.SMEM((1, 2), jnp.float32)],
    )(logits, target)


@jax.jit
def kernel(contrast_logits, contrast_target):
    return _tc_loss(contrast_logits, contrast_target)[0, 0]


# speedup vs baseline: 1.4879x; 1.4879x over previous
import functools

import jax
import jax.numpy as jnp
from jax import lax
from jax.experimental import pallas as pl
from jax.experimental.pallas import tpu as pltpu
from jax.experimental.pallas import tpu_sc as plsc

N = 32768
C = 2048
NC, NS, L = 2, 16, 16
NW = NC * NS
CB = 128
TPB = 32
BR = 512

N_TC = 11264
N_SC = N - N_TC
PER_W = N_SC // NW
NBATCH = PER_W // TPB
G_TC = N_TC // BR
TC_OFF = N_SC // BR


def _sc_partials(logits, target):
    mesh = plsc.VectorSubcoreMesh(core_axis_name="c", subcore_axis_name="s")

    @functools.partial(
        pl.kernel,
        out_type=jax.ShapeDtypeStruct((NW * 2 * L,), jnp.float32),
        mesh=mesh,
        compiler_params=pltpu.CompilerParams(
            use_tc_tiling_on_sc=True, needs_layout_passes=False
        ),
        scratch_types=[
            pltpu.VMEM((PER_W,), jnp.int32),
            pltpu.VMEM((2, TPB, 8, CB), jnp.float32),
            pltpu.VMEM((2 * L,), jnp.float32),
            pltpu.SemaphoreType.DMA,
        ],
    )
    def kern(logits_hbm, tgt_hbm, out_hbm, tgt_v, gat_v, acc_v, sem):
        wid = lax.axis_index("s") * NC + lax.axis_index("c")
        base = wid * PER_W

        pltpu.sync_copy(tgt_hbm.at[pl.ds(base, PER_W)], tgt_v)

        lane = lax.iota(jnp.int32, L)

        def fire(b):
            p = b & 1
            for q in range(TPB // L):
                t16 = tgt_v[pl.ds(b * TPB + q * L, L)]
                cb16 = jnp.where(t16 >= 0, t16, 0) >> 7
                for l in range(L):
                    e = q * L + l
                    row = base + b * TPB + e
                    rowa = pl.multiple_of((row >> 3) << 3, 8)
                    cstart = pl.multiple_of(cb16[l] << 7, CB)
                    pltpu.make_async_copy(
                        logits_hbm.at[pl.ds(rowa, 8), pl.ds(cstart, CB)],
                        gat_v.at[p, e],
                        sem,
                    ).start()

        def drain():
            for _ in range(4):
                pltpu.make_async_copy(
                    logits_hbm.at[pl.ds(0, 8 * (TPB // 4)), pl.ds(0, CB)],
                    gat_v.at[0, pl.ds(0, TPB // 4)],
                    sem,
                ).wait()

        def extract(b, acc):
            a_sq, a_m = acc
            p = b & 1
            for q in range(TPB // L):
                t16 = tgt_v[pl.ds(b * TPB + q * L, L)]
                safe = jnp.where(t16 >= 0, t16, 0)
                m16 = jnp.where(t16 >= 0, 1.0, 0.0).astype(jnp.float32)
                slot = q * L + lane
                sub = (base + b * TPB + slot) & 7
                col = safe & (CB - 1)
                g16 = plsc.load_gather(gat_v.at[p], [slot, sub, col])
                d = 1.0 - g16
                a_sq = a_sq + d * d * m16
                a_m = a_m + m16
            return a_sq, a_m

        fire(0)

        def body(b, acc):
            fire(b + 1)
            drain()
            return extract(b, acc)

        acc_sq, acc_m = lax.fori_loop(
            0, NBATCH - 1, body,
            (jnp.zeros((L,), jnp.float32), jnp.zeros((L,), jnp.float32)),
            unroll=False,
        )
        drain()
        acc_sq, acc_m = extract(NBATCH - 1, (acc_sq, acc_m))

        acc_v[pl.ds(0, L)] = acc_sq
        acc_v[pl.ds(L, L)] = acc_m
        pltpu.sync_copy(acc_v.at[pl.ds(0, L)], out_hbm.at[pl.ds(wid * L, L)])
        pltpu.sync_copy(
            acc_v.at[pl.ds(L, L)], out_hbm.at[pl.ds(NW * L + wid * L, L)]
        )

    return kern(logits, target)


def _tc_partials(logits, target):
    def body(x_ref, t_ref, o_ref, acc):
        i = pl.program_id(0)

        @pl.when(i == 0)
        def _():
            acc[0, 0] = 0.0
            acc[0, 1] = 0.0

        x = x_ref[...]
        tb = t_ref[...]
        cols = jax.lax.broadcasted_iota(jnp.int32, (BR, C), 1)
        safe = jnp.where(tb >= 0, tb, -2)
        m = cols == safe[:, None]
        d = 1.0 - x
        acc[0, 0] += jnp.sum(jnp.where(m, d * d, 0.0))
        acc[0, 1] += jnp.sum(jnp.where(tb >= 0, 1.0, 0.0))

        @pl.when(i == G_TC - 1)
        def _():
            o_ref[...] = jnp.stack(
                [acc[0, 0], acc[0, 1]]
            ).reshape(1, 2)

    return pl.pallas_call(
        body,
        grid=(G_TC,),
        in_specs=[
            pl.BlockSpec((BR, C), lambda i: (i + TC_OFF, 0)),
            pl.BlockSpec((BR,), lambda i: (i + TC_OFF,)),
        ],
        out_specs=pl.BlockSpec((1, 2), lambda i: (0, 0)),
        out_shape=jax.ShapeDtypeStruct((1, 2), jnp.float32),
        scratch_shapes=[pltpu.SMEM((1, 2), jnp.float32)],
    )(logits, target)


def _finalize(sc_p, tc_p):
    def body(p_ref, a_ref, o_ref):
        p = p_ref[...]
        a = a_ref[...]
        s = jnp.sum(p[0:4]) + a[0, 0]
        m = jnp.sum(p[4:8]) + a[0, 1]
        o_ref[...] = jnp.full((1, 1), s / m, jnp.float32)

    return pl.pallas_call(
        body,
        out_shape=jax.ShapeDtypeStruct((1, 1), jnp.float32),
    )(sc_p, tc_p)


@jax.jit
def kernel(contrast_logits, contrast_target):
    sc_p = _sc_partials(contrast_logits, contrast_target)
    tc_p = _tc_partials(contrast_logits, contrast_target)
    loss = _finalize(sc_p.reshape(8, 128), tc_p)
    return loss[0, 0]


# speedup vs baseline: 1.9361x; 1.3012x over previous
import functools

import jax
import jax.numpy as jnp
from jax import lax
from jax.experimental import pallas as pl
from jax.experimental.pallas import tpu as pltpu
from jax.experimental.pallas import tpu_sc as plsc

N = 32768
C = 2048
NC, NS, L = 2, 16, 16
NW = NC * NS
PER_W = N // NW
CB = 128
NCB = C // CB
TPB = 32
NBATCH = PER_W // TPB


def _sc_partials(logits, target):
    mesh = plsc.VectorSubcoreMesh(core_axis_name="c", subcore_axis_name="s")

    @functools.partial(
        pl.kernel,
        out_type=jax.ShapeDtypeStruct((NW * 2 * L,), jnp.float32),
        mesh=mesh,
        compiler_params=pltpu.CompilerParams(
            use_tc_tiling_on_sc=True, needs_layout_passes=False
        ),
        scratch_types=[
            pltpu.VMEM((8 + PER_W,), jnp.int32),
            pltpu.VMEM((2, TPB, 8, CB), jnp.float32),
            pltpu.VMEM((2 * L,), jnp.float32),
            pltpu.SemaphoreType.DMA,
        ],
    )
    def kern(logits_hbm, tgt_hbm, out_hbm, tgt_v, gat_v, acc_v, sem):
        wid = lax.axis_index("s") * NC + lax.axis_index("c")
        base = wid * PER_W

        tgt_v[pl.ds(0, L)] = jnp.zeros((L,), jnp.int32)
        pltpu.sync_copy(tgt_hbm.at[pl.ds(base, PER_W)], tgt_v.at[pl.ds(8, PER_W)])

        lane = lax.iota(jnp.int32, L)
        lanepos = lane & 7

        def cb_of(t):
            return jnp.where(t >= 0, t, 0) >> 7

        def dists(b, q):
            off = 8 + b * TPB + q * L
            cb16 = cb_of(tgt_v[pl.ds(off, L)])
            dist = jnp.zeros((L,), jnp.int32)
            for d in range(1, 8):
                cbs = cb_of(tgt_v[pl.ds(off - d, L)])
                match = (cb16 == cbs) & (lanepos >= d)
                dist = jnp.where(match, d, dist)
            return dist

        def fire(b):
            p = b & 1
            nfetch = jnp.int32(0)
            for q in range(TPB // L):
                t16 = tgt_v[pl.ds(8 + b * TPB + q * L, L)]
                cb16 = cb_of(t16)
                dist = dists(b, q)
                cnt = plsc.all_reduce_population_count(dist == 0)
                nfetch = nfetch + cnt[0]
                for l in range(L):
                    e = q * L + l

                    @pl.when(dist[l] == 0)
                    def _():
                        row = base + b * TPB + e
                        rowa = pl.multiple_of((row >> 3) << 3, 8)
                        cstart = pl.multiple_of(cb16[l] << 7, CB)
                        pltpu.make_async_copy(
                            logits_hbm.at[pl.ds(rowa, 8), pl.ds(cstart, CB)],
                            gat_v.at[p, e],
                            sem,
                        ).start()

            return nfetch

        def drain(k):
            def w(_, carry):
                pltpu.make_async_copy(
                    logits_hbm.at[pl.ds(0, 8), pl.ds(0, CB)],
                    gat_v.at[0, 0],
                    sem,
                ).wait()
                return carry

            lax.fori_loop(0, k, w, 0, unroll=False)

        def extract(b, acc):
            a_sq, a_m = acc
            p = b & 1
            for q in range(TPB // L):
                t16 = tgt_v[pl.ds(8 + b * TPB + q * L, L)]
                safe = jnp.where(t16 >= 0, t16, 0)
                m16 = jnp.where(t16 >= 0, 1.0, 0.0).astype(jnp.float32)
                dist = dists(b, q)
                slot = q * L + lane - dist
                sub = (base + b * TPB + q * L + lane) & 7
                col = safe & (CB - 1)
                g16 = plsc.load_gather(gat_v.at[p], [slot, sub, col])
                d = 1.0 - g16
                a_sq = a_sq + d * d * m16
                a_m = a_m + m16
            return a_sq, a_m

        k0 = fire(0)

        def body(b, carry):
            a_sq, a_m, kprev = carry
            knext = fire(b + 1)
            drain(kprev)
            a_sq, a_m = extract(b, (a_sq, a_m))
            return a_sq, a_m, knext

        acc_sq, acc_m, klast = lax.fori_loop(
            0, NBATCH - 1, body,
            (jnp.zeros((L,), jnp.float32), jnp.zeros((L,), jnp.float32), k0),
            unroll=False,
        )
        drain(klast)
        acc_sq, acc_m = extract(NBATCH - 1, (acc_sq, acc_m))

        acc_v[pl.ds(0, L)] = acc_sq
        acc_v[pl.ds(L, L)] = acc_m
        pltpu.sync_copy(acc_v.at[pl.ds(0, L)], out_hbm.at[pl.ds(wid * L, L)])
        pltpu.sync_copy(
            acc_v.at[pl.ds(L, L)], out_hbm.at[pl.ds(NW * L + wid * L, L)]
        )

    return kern(logits, target)


def _tc_finalize(partials):
    def body(p_ref, o_ref):
        p = p_ref[...]
        s = jnp.sum(p[0:4])
        m = jnp.sum(p[4:8])
        o_ref[...] = jnp.full((1, 1), s / m, jnp.float32)

    return pl.pallas_call(
        body,
        out_shape=jax.ShapeDtypeStruct((1, 1), jnp.float32),
    )(partials)


@jax.jit
def kernel(contrast_logits, contrast_target):
    partials = _sc_partials(contrast_logits, contrast_target)
    loss = _tc_finalize(partials.reshape(8, 128))
    return loss[0, 0]
